# baseline (speedup 1.0000x reference)
import jax
import jax.numpy as jnp
from jax import lax
from jax.experimental import pallas as pl
from jax.experimental.pallas import tpu as pltpu

B = 64
V = 100000
D = 768
NB = 16
NGRAM = 4
SEED = 42
BIAS = 2.0
TOP_P = 0.9

NPAD = 131072
ROWS = 1024
COLS = 128
LOGN = 17

_I32 = jnp.int32


def _tf_scalar(k0, k1, x0, x1):
    ks2 = k0 ^ k1 ^ _I32(0x1BD11BDA)
    rots = [[13, 15, 26, 6], [17, 29, 16, 24], [13, 15, 26, 6],
            [17, 29, 16, 24], [13, 15, 26, 6]]
    keys = [(k1, ks2), (ks2, k0), (k0, k1), (k1, ks2), (ks2, k0)]
    x0 = x0 + k0
    x1 = x1 + k1
    for i in range(5):
        for r in rots[i]:
            x0 = x0 + x1
            x1 = ((x1 << r) | lax.shift_right_logical(x1, 32 - r)) ^ x0
        x0 = x0 + keys[i][0]
        x1 = x1 + keys[i][1] + _I32(i + 1)
    return x0, x1


def _main_kernel(keymat_ref, scores_ref, out_ref, keys_ref):
    b = pl.program_id(0)
    k2_0 = keymat_ref[b, 0]
    k2_1 = keymat_ref[b, 1]
    sk0 = keymat_ref[b, 2]
    sk1 = keymat_ref[b, 3]

    rr = lax.broadcasted_iota(_I32, (ROWS, COLS), 0)
    cc = lax.broadcasted_iota(_I32, (ROWS, COLS), 1)
    J = rr * COLS + cc

    zv = jnp.zeros((ROWS, COLS), _I32)
    y0, y1 = _tf_scalar(k2_0, k2_1, zv, J)
    green = ((y0 ^ y1) & 1).astype(jnp.float32)

    logits = scores_ref[0] + jnp.float32(BIAS) * green
    m = jnp.max(logits)
    e = jnp.exp(logits - m)
    z = jnp.sum(e)
    probs = e / z

    okeys = pltpu.bitcast(probs, _I32)
    keys_ref[...] = okeys

    def phase_body(p, _):
        k = lax.shift_left(_I32(1), p + 1)
        dk = (J & k) == 0

        def sub_stage(q, __):
            d = lax.shift_left(_I32(1), p - q)
            dd = lax.shift_right_logical(d, 7)
            up = (J & d) == 0
            sel = up == dk
            ka = keys_ref[...]
            kp = jnp.where(up, pltpu.roll(ka, ROWS - dd, 0),
                           pltpu.roll(ka, dd, 0))
            keys_ref[...] = jnp.where(sel, jnp.maximum(ka, kp),
                                      jnp.minimum(ka, kp))
            return 0

        def lane_stage(q, __):
            d = lax.shift_left(_I32(1), jnp.minimum(p, 6) - q)
            up = (J & d) == 0
            sel = up == dk
            ka = keys_ref[...]
            kp = jnp.where(up, pltpu.roll(ka, COLS - d, 1),
                           pltpu.roll(ka, d, 1))
            keys_ref[...] = jnp.where(sel, jnp.maximum(ka, kp),
                                      jnp.minimum(ka, kp))
            return 0

        nsub = jnp.maximum(p - 6, 0)
        lax.fori_loop(0, nsub, sub_stage, 0)
        return lax.fori_loop(0, jnp.minimum(p, 6) + 1, lane_stage, 0)

    lax.fori_loop(0, LOGN, phase_body, 0)

    sp = pltpu.bitcast(keys_ref[...], jnp.float32)

    s1 = sp
    for t in (1, 2, 4, 8, 16, 32, 64):
        s1 = s1 + jnp.where(cc >= t, pltpu.roll(s1, t, 1), 0.0)
    rtot = s1[:, COLS - 1:COLS]
    rc = lax.broadcasted_iota(_I32, (ROWS, 1), 0)
    c = rtot
    for t in (1, 2, 4, 8, 16, 32, 64, 128, 256, 512):
        c = c + jnp.where(rc >= t, pltpu.roll(c, t, 0), 0.0)
    cum = s1 + (c - rtot)

    cutoff = jnp.sum((cum < jnp.float32(TOP_P)).astype(_I32))
    kept = J <= cutoff
    s_tot = jnp.sum(jnp.where(kept, sp, 0.0))

    u0, u1 = _tf_scalar(sk0, sk1, zv, J)
    bits = u0 ^ u1
    fl = pltpu.bitcast(lax.shift_right_logical(bits, 9) | _I32(0x3F800000),
                       jnp.float32) - jnp.float32(1.0)
    tiny = jnp.float32(jnp.finfo(jnp.float32).tiny)
    u = jnp.maximum(tiny, fl * (jnp.float32(1.0) - tiny) + tiny)
    gum = -jnp.log(-jnp.log(u))

    val = jnp.where(kept, jnp.log(sp / s_tot), -jnp.inf) + gum
    vmax = jnp.max(val)
    win = jnp.min(jnp.where(val == vmax, J, _I32(2 ** 30)))

    kstar = jnp.sum(jnp.where(J == win, keys_ref[...], 0))
    first = jnp.sum((okeys > kstar).astype(_I32))
    occ = (win - first).astype(jnp.float32)
    eq = (okeys == kstar).astype(jnp.float32)
    e1 = eq
    for t in (1, 2, 4, 8, 16, 32, 64):
        e1 = e1 + jnp.where(cc >= t, pltpu.roll(e1, t, 1), 0.0)
    etot = e1[:, COLS - 1:COLS]
    ec = etot
    for t in (1, 2, 4, 8, 16, 32, 64, 128, 256, 512):
        ec = ec + jnp.where(rc >= t, pltpu.roll(ec, t, 0), 0.0)
    ecum_excl = e1 + (ec - etot) - eq
    next_token = jnp.sum(jnp.where((eq > 0) & (ecum_excl == occ), J, 0))

    out_ref[0] = jnp.where(J == next_token, jnp.float32(1e5), jnp.float32(1e-5))


def _tfv(k0, k1, x0, x1):
    u = jnp.uint32
    ks2 = k0 ^ k1 ^ u(0x1BD11BDA)
    rots = [[13, 15, 26, 6], [17, 29, 16, 24], [13, 15, 26, 6],
            [17, 29, 16, 24], [13, 15, 26, 6]]
    keys = [(k1, ks2), (ks2, k0), (k0, k1), (k1, ks2), (ks2, k0)]
    x0 = x0 + k0
    x1 = x1 + k1
    for i in range(5):
        for r in rots[i]:
            x0 = x0 + x1
            x1 = ((x1 << u(r)) | (x1 >> u(32 - r))) ^ x0
        x0 = x0 + keys[i][0]
        x1 = x1 + keys[i][1] + u(i + 1)
    return x0, x1


@jax.jit
def kernel(input_ids, scores, embed_table, random_vectors):
    tails = input_ids[:, -(NGRAM - 1):]
    scores_p = jnp.pad(scores, ((0, 0), (0, NPAD - V)),
                       constant_values=-1e30).reshape(B, ROWS, COLS)

    input_vec = jnp.take(embed_table, tails, axis=0).mean(axis=1)
    projections = input_vec @ random_vectors.T
    binary = (projections > 0).astype(jnp.int32)
    simhash_seed = SEED + jnp.sum(
        binary * (2 ** jnp.arange(NB, dtype=jnp.int32)), axis=1)
    seed_u = simhash_seed.astype(jnp.uint32)
    z = jnp.zeros_like(seed_u)
    gk0, gk1 = _tfv(jnp.uint32(0), jnp.uint32(SEED), z, seed_u)
    k2_0, k2_1 = _tfv(gk0, gk1, z, jnp.ones_like(seed_u))
    bidx = jnp.arange(B, dtype=jnp.uint32)
    sk0, sk1 = _tfv(jnp.uint32(0), jnp.uint32(123), z, bidx)
    keymat = lax.bitcast_convert_type(
        jnp.stack([k2_0, k2_1, sk0, sk1], axis=1), jnp.int32)

    out = pl.pallas_call(
        _main_kernel,
        grid=(B,),
        in_specs=[
            pl.BlockSpec(memory_space=pltpu.SMEM),
            pl.BlockSpec((1, ROWS, COLS), lambda b: (b, 0, 0)),
        ],
        out_specs=pl.BlockSpec((1, ROWS, COLS), lambda b: (b, 0, 0)),
        out_shape=jax.ShapeDtypeStruct((B, ROWS, COLS), jnp.float32),
        scratch_shapes=[
            pltpu.VMEM((ROWS, COLS), _I32),
        ],
    )(keymat, scores_p)

    return out.reshape(B, NPAD)[:, :V]


# speedup vs baseline: 1.1296x; 1.1296x over previous
import jax
import jax.numpy as jnp
from jax import lax
from jax.experimental import pallas as pl
from jax.experimental.pallas import tpu as pltpu

B = 64
V = 100000
D = 768
NB = 16
NGRAM = 4
SEED = 42
BIAS = 2.0
TOP_P = 0.9

NPAD = 131072
ROWS = 1024
COLS = 128
LOGN = 17

_I32 = jnp.int32


def _tf_scalar(k0, k1, x0, x1):
    ks2 = k0 ^ k1 ^ _I32(0x1BD11BDA)
    rots = [[13, 15, 26, 6], [17, 29, 16, 24], [13, 15, 26, 6],
            [17, 29, 16, 24], [13, 15, 26, 6]]
    keys = [(k1, ks2), (ks2, k0), (k0, k1), (k1, ks2), (ks2, k0)]
    x0 = x0 + k0
    x1 = x1 + k1
    for i in range(5):
        for r in rots[i]:
            x0 = x0 + x1
            x1 = ((x1 << r) | lax.shift_right_logical(x1, 32 - r)) ^ x0
        x0 = x0 + keys[i][0]
        x1 = x1 + keys[i][1] + _I32(i + 1)
    return x0, x1


def _main_kernel(keymat_ref, scores_ref, out_ref, keys_ref):
    b = pl.program_id(0)
    k2_0 = keymat_ref[b, 0]
    k2_1 = keymat_ref[b, 1]
    sk0 = keymat_ref[b, 2]
    sk1 = keymat_ref[b, 3]

    rr = lax.broadcasted_iota(_I32, (ROWS, COLS), 0)
    cc = lax.broadcasted_iota(_I32, (ROWS, COLS), 1)
    J = rr * COLS + cc

    zv = jnp.zeros((ROWS, COLS), _I32)
    y0, y1 = _tf_scalar(k2_0, k2_1, zv, J)
    green = ((y0 ^ y1) & 1).astype(jnp.float32)

    logits = scores_ref[0] + jnp.float32(BIAS) * green
    m = jnp.max(logits)
    e = jnp.exp(logits - m)
    z = jnp.sum(e)
    probs = e / z

    okeys = pltpu.bitcast(probs, _I32)
    keys_ref[...] = okeys

    def _lane_cx(ka, d, dk):
        up = (J & d) == 0
        sel = up == dk
        kp = jnp.where(up, pltpu.roll(ka, COLS - d, 1),
                       pltpu.roll(ka, d, 1))
        return jnp.where(sel, jnp.maximum(ka, kp), jnp.minimum(ka, kp))

    ka0 = keys_ref[...]
    for p in range(7):
        dk = (J & (1 << (p + 1))) == 0
        for q in range(p + 1):
            ka0 = _lane_cx(ka0, 1 << (p - q), dk)
    keys_ref[...] = ka0

    def phase_body(p, _):
        k = lax.shift_left(_I32(1), p + 1)
        dk = (J & k) == 0

        def sub_stage(q, __):
            d = lax.shift_left(_I32(1), p - q)
            dd = lax.shift_right_logical(d, 7)
            up = (J & d) == 0
            sel = up == dk
            ka = keys_ref[...]
            kp = jnp.where(up, pltpu.roll(ka, ROWS - dd, 0),
                           pltpu.roll(ka, dd, 0))
            keys_ref[...] = jnp.where(sel, jnp.maximum(ka, kp),
                                      jnp.minimum(ka, kp))
            return 0

        lax.fori_loop(0, p - 6, sub_stage, 0)

        ka = keys_ref[...]
        for d in (64, 32, 16, 8, 4, 2, 1):
            ka = _lane_cx(ka, d, dk)
        keys_ref[...] = ka
        return 0

    lax.fori_loop(7, LOGN, phase_body, 0)

    sp = pltpu.bitcast(keys_ref[...], jnp.float32)

    s1 = sp
    for t in (1, 2, 4, 8, 16, 32, 64):
        s1 = s1 + jnp.where(cc >= t, pltpu.roll(s1, t, 1), 0.0)
    rtot = s1[:, COLS - 1:COLS]
    rc = lax.broadcasted_iota(_I32, (ROWS, 1), 0)
    c = rtot
    for t in (1, 2, 4, 8, 16, 32, 64, 128, 256, 512):
        c = c + jnp.where(rc >= t, pltpu.roll(c, t, 0), 0.0)
    cum = s1 + (c - rtot)

    cutoff = jnp.sum((cum < jnp.float32(TOP_P)).astype(_I32))
    kept = J <= cutoff
    s_tot = jnp.sum(jnp.where(kept, sp, 0.0))

    u0, u1 = _tf_scalar(sk0, sk1, zv, J)
    bits = u0 ^ u1
    fl = pltpu.bitcast(lax.shift_right_logical(bits, 9) | _I32(0x3F800000),
                       jnp.float32) - jnp.float32(1.0)
    tiny = jnp.float32(jnp.finfo(jnp.float32).tiny)
    u = jnp.maximum(tiny, fl * (jnp.float32(1.0) - tiny) + tiny)
    gum = -jnp.log(-jnp.log(u))

    val = jnp.where(kept, jnp.log(sp / s_tot), -jnp.inf) + gum
    vmax = jnp.max(val)
    win = jnp.min(jnp.where(val == vmax, J, _I32(2 ** 30)))

    kstar = jnp.sum(jnp.where(J == win, keys_ref[...], 0))
    first = jnp.sum((okeys > kstar).astype(_I32))
    occ = (win - first).astype(jnp.float32)
    eq = (okeys == kstar).astype(jnp.float32)
    e1 = eq
    for t in (1, 2, 4, 8, 16, 32, 64):
        e1 = e1 + jnp.where(cc >= t, pltpu.roll(e1, t, 1), 0.0)
    etot = e1[:, COLS - 1:COLS]
    ec = etot
    for t in (1, 2, 4, 8, 16, 32, 64, 128, 256, 512):
        ec = ec + jnp.where(rc >= t, pltpu.roll(ec, t, 0), 0.0)
    ecum_excl = e1 + (ec - etot) - eq
    next_token = jnp.sum(jnp.where((eq > 0) & (ecum_excl == occ), J, 0))

    out_ref[0] = jnp.where(J == next_token, jnp.float32(1e5), jnp.float32(1e-5))


def _tfv(k0, k1, x0, x1):
    u = jnp.uint32
    ks2 = k0 ^ k1 ^ u(0x1BD11BDA)
    rots = [[13, 15, 26, 6], [17, 29, 16, 24], [13, 15, 26, 6],
            [17, 29, 16, 24], [13, 15, 26, 6]]
    keys = [(k1, ks2), (ks2, k0), (k0, k1), (k1, ks2), (ks2, k0)]
    x0 = x0 + k0
    x1 = x1 + k1
    for i in range(5):
        for r in rots[i]:
            x0 = x0 + x1
            x1 = ((x1 << u(r)) | (x1 >> u(32 - r))) ^ x0
        x0 = x0 + keys[i][0]
        x1 = x1 + keys[i][1] + u(i + 1)
    return x0, x1


@jax.jit
def kernel(input_ids, scores, embed_table, random_vectors):
    tails = input_ids[:, -(NGRAM - 1):]
    scores_p = jnp.pad(scores, ((0, 0), (0, NPAD - V)),
                       constant_values=-1e30).reshape(B, ROWS, COLS)

    input_vec = jnp.take(embed_table, tails, axis=0).mean(axis=1)
    projections = input_vec @ random_vectors.T
    binary = (projections > 0).astype(jnp.int32)
    simhash_seed = SEED + jnp.sum(
        binary * (2 ** jnp.arange(NB, dtype=jnp.int32)), axis=1)
    seed_u = simhash_seed.astype(jnp.uint32)
    z = jnp.zeros_like(seed_u)
    gk0, gk1 = _tfv(jnp.uint32(0), jnp.uint32(SEED), z, seed_u)
    k2_0, k2_1 = _tfv(gk0, gk1, z, jnp.ones_like(seed_u))
    bidx = jnp.arange(B, dtype=jnp.uint32)
    sk0, sk1 = _tfv(jnp.uint32(0), jnp.uint32(123), z, bidx)
    keymat = lax.bitcast_convert_type(
        jnp.stack([k2_0, k2_1, sk0, sk1], axis=1), jnp.int32)

    out = pl.pallas_call(
        _main_kernel,
        grid=(B,),
        in_specs=[
            pl.BlockSpec(memory_space=pltpu.SMEM),
            pl.BlockSpec((1, ROWS, COLS), lambda b: (b, 0, 0)),
        ],
        out_specs=pl.BlockSpec((1, ROWS, COLS), lambda b: (b, 0, 0)),
        out_shape=jax.ShapeDtypeStruct((B, ROWS, COLS), jnp.float32),
        scratch_shapes=[
            pltpu.VMEM((ROWS, COLS), _I32),
        ],
    )(keymat, scores_p)

    return out.reshape(B, NPAD)[:, :V]


# speedup vs baseline: 1.2606x; 1.1159x over previous
import jax
import jax.numpy as jnp
from jax import lax
from jax.experimental import pallas as pl
from jax.experimental.pallas import tpu as pltpu

B = 64
V = 100000
D = 768
NB = 16
NGRAM = 4
SEED = 42
BIAS = 2.0
TOP_P = 0.9

NPAD = 131072
ROWS = 1024
COLS = 128
LOGN = 17

_I32 = jnp.int32


def _tf_scalar(k0, k1, x0, x1):
    ks2 = k0 ^ k1 ^ _I32(0x1BD11BDA)
    rots = [[13, 15, 26, 6], [17, 29, 16, 24], [13, 15, 26, 6],
            [17, 29, 16, 24], [13, 15, 26, 6]]
    keys = [(k1, ks2), (ks2, k0), (k0, k1), (k1, ks2), (ks2, k0)]
    x0 = x0 + k0
    x1 = x1 + k1
    for i in range(5):
        for r in rots[i]:
            x0 = x0 + x1
            x1 = ((x1 << r) | lax.shift_right_logical(x1, 32 - r)) ^ x0
        x0 = x0 + keys[i][0]
        x1 = x1 + keys[i][1] + _I32(i + 1)
    return x0, x1


def _main_kernel(keymat_ref, scores_ref, out_ref, keys_ref):
    b = pl.program_id(0)
    k2_0 = keymat_ref[b, 0]
    k2_1 = keymat_ref[b, 1]
    sk0 = keymat_ref[b, 2]
    sk1 = keymat_ref[b, 3]

    rr = lax.broadcasted_iota(_I32, (ROWS, COLS), 0)
    cc = lax.broadcasted_iota(_I32, (ROWS, COLS), 1)
    J = rr * COLS + cc

    zv = jnp.zeros((ROWS, COLS), _I32)
    y0, y1 = _tf_scalar(k2_0, k2_1, zv, J)
    green = ((y0 ^ y1) & 1).astype(jnp.float32)

    logits = scores_ref[0] + jnp.float32(BIAS) * green
    m = jnp.max(logits)
    e = jnp.exp(logits - m)
    z = jnp.sum(e)
    probs = e / z

    okeys = pltpu.bitcast(probs, _I32)

    def _lane_cx(ka, d, dk):
        up = (J & d) == 0
        sel = up == dk
        kp = jnp.where(up, pltpu.roll(ka, COLS - d, 1),
                       pltpu.roll(ka, d, 1))
        return jnp.where(sel, jnp.maximum(ka, kp), jnp.minimum(ka, kp))

    ka0 = okeys
    for p in range(7):
        dk = (J & (1 << (p + 1))) == 0
        for q in range(p + 1):
            ka0 = _lane_cx(ka0, 1 << (p - q), dk)
    keys_ref[...] = ka0

    def phase_body(p, _):
        k = lax.shift_left(_I32(1), p + 1)
        dk = (J & k) == 0

        def _sub_cx(ka, d):
            dd = lax.shift_right_logical(d, 7)
            up = (J & d) == 0
            sel = up == dk
            kp = jnp.where(up, pltpu.roll(ka, ROWS - dd, 0),
                           pltpu.roll(ka, dd, 0))
            return jnp.where(sel, jnp.maximum(ka, kp), jnp.minimum(ka, kp))

        def pair_stage(i, __):
            d1 = lax.shift_left(_I32(1), p - 2 * i)
            ka = _sub_cx(keys_ref[...], d1)
            keys_ref[...] = _sub_cx(ka, lax.shift_right_logical(d1, 1))
            return 0

        def sub_stage(q, __):
            d = lax.shift_left(_I32(1), p - q)
            keys_ref[...] = _sub_cx(keys_ref[...], d)
            return 0

        nsub = p - 6
        npairs = lax.shift_right_logical(nsub, 1)
        lax.fori_loop(0, npairs, pair_stage, 0)
        lax.fori_loop(2 * npairs, nsub, sub_stage, 0)

        ka = keys_ref[...]
        for d in (64, 32, 16, 8, 4, 2, 1):
            ka = _lane_cx(ka, d, dk)
        keys_ref[...] = ka
        return 0

    lax.fori_loop(7, LOGN, phase_body, 0)

    sp = pltpu.bitcast(keys_ref[...], jnp.float32)

    s1 = sp
    for t in (1, 2, 4, 8, 16, 32, 64):
        s1 = s1 + jnp.where(cc >= t, pltpu.roll(s1, t, 1), 0.0)
    rtot = s1[:, COLS - 1:COLS]
    rc = lax.broadcasted_iota(_I32, (ROWS, 1), 0)
    c = rtot
    for t in (1, 2, 4, 8, 16, 32, 64, 128, 256, 512):
        c = c + jnp.where(rc >= t, pltpu.roll(c, t, 0), 0.0)
    cum = s1 + (c - rtot)

    cutoff = jnp.sum((cum < jnp.float32(TOP_P)).astype(_I32))
    kept = J <= cutoff
    s_tot = jnp.sum(jnp.where(kept, sp, 0.0))

    u0, u1 = _tf_scalar(sk0, sk1, zv, J)
    bits = u0 ^ u1
    fl = pltpu.bitcast(lax.shift_right_logical(bits, 9) | _I32(0x3F800000),
                       jnp.float32) - jnp.float32(1.0)
    tiny = jnp.float32(jnp.finfo(jnp.float32).tiny)
    u = jnp.maximum(tiny, fl * (jnp.float32(1.0) - tiny) + tiny)
    gum = -jnp.log(-jnp.log(u))

    val = jnp.where(kept, jnp.log(sp / s_tot), -jnp.inf) + gum
    vmax = jnp.max(val)
    win = jnp.min(jnp.where(val == vmax, J, _I32(2 ** 30)))

    kstar = jnp.sum(jnp.where(J == win, keys_ref[...], 0))
    first = jnp.sum((okeys > kstar).astype(_I32))
    occ = (win - first).astype(jnp.float32)
    eq = (okeys == kstar).astype(jnp.float32)
    e1 = eq
    for t in (1, 2, 4, 8, 16, 32, 64):
        e1 = e1 + jnp.where(cc >= t, pltpu.roll(e1, t, 1), 0.0)
    etot = e1[:, COLS - 1:COLS]
    ec = etot
    for t in (1, 2, 4, 8, 16, 32, 64, 128, 256, 512):
        ec = ec + jnp.where(rc >= t, pltpu.roll(ec, t, 0), 0.0)
    ecum_excl = e1 + (ec - etot) - eq
    next_token = jnp.sum(jnp.where((eq > 0) & (ecum_excl == occ), J, 0))

    out_ref[0] = jnp.where(J == next_token, jnp.float32(1e5), jnp.float32(1e-5))


def _tfv(k0, k1, x0, x1):
    u = jnp.uint32
    ks2 = k0 ^ k1 ^ u(0x1BD11BDA)
    rots = [[13, 15, 26, 6], [17, 29, 16, 24], [13, 15, 26, 6],
            [17, 29, 16, 24], [13, 15, 26, 6]]
    keys = [(k1, ks2), (ks2, k0), (k0, k1), (k1, ks2), (ks2, k0)]
    x0 = x0 + k0
    x1 = x1 + k1
    for i in range(5):
        for r in rots[i]:
            x0 = x0 + x1
            x1 = ((x1 << u(r)) | (x1 >> u(32 - r))) ^ x0
        x0 = x0 + keys[i][0]
        x1 = x1 + keys[i][1] + u(i + 1)
    return x0, x1


@jax.jit
def kernel(input_ids, scores, embed_table, random_vectors):
    tails = input_ids[:, -(NGRAM - 1):]
    scores_p = jnp.pad(scores, ((0, 0), (0, NPAD - V)),
                       constant_values=-1e30).reshape(B, ROWS, COLS)

    input_vec = jnp.take(embed_table, tails, axis=0).mean(axis=1)
    projections = input_vec @ random_vectors.T
    binary = (projections > 0).astype(jnp.int32)
    simhash_seed = SEED + jnp.sum(
        binary * (2 ** jnp.arange(NB, dtype=jnp.int32)), axis=1)
    seed_u = simhash_seed.astype(jnp.uint32)
    z = jnp.zeros_like(seed_u)
    gk0, gk1 = _tfv(jnp.uint32(0), jnp.uint32(SEED), z, seed_u)
    k2_0, k2_1 = _tfv(gk0, gk1, z, jnp.ones_like(seed_u))
    bidx = jnp.arange(B, dtype=jnp.uint32)
    sk0, sk1 = _tfv(jnp.uint32(0), jnp.uint32(123), z, bidx)
    keymat = lax.bitcast_convert_type(
        jnp.stack([k2_0, k2_1, sk0, sk1], axis=1), jnp.int32)

    out = pl.pallas_call(
        _main_kernel,
        grid=(B,),
        in_specs=[
            pl.BlockSpec(memory_space=pltpu.SMEM),
            pl.BlockSpec((1, ROWS, COLS), lambda b: (b, 0, 0)),
        ],
        out_specs=pl.BlockSpec((1, ROWS, COLS), lambda b: (b, 0, 0)),
        out_shape=jax.ShapeDtypeStruct((B, ROWS, COLS), jnp.float32),
        scratch_shapes=[
            pltpu.VMEM((ROWS, COLS), _I32),
        ],
    )(keymat, scores_p)

    return out.reshape(B, NPAD)[:, :V]
